# baseline (speedup 1.0000x reference)
import functools

import jax
import jax.numpy as jnp
from jax import lax
from jax.experimental import pallas as pl
from jax.experimental.pallas import tpu as pltpu
from jax.experimental.pallas import tpu_sc as plsc

N_NODES = 10000
E_EDGES = 320000
D = 128

NC = 2
NS = 16
NW = NC * NS

CHUNK = 128
N_CHUNKS = 80
EPW = CHUNK * N_CHUNKS
E_PAD = EPW * NW
N_PAD = 10112
ACC_ROWS = 10008
RPT = 632
LAST_RPT = ACC_ROWS - 15 * RPT
OUT_W = 16

_mesh = plsc.VectorSubcoreMesh(core_axis_name="c", subcore_axis_name="s")


C0_CHUNKS = 56
C1_CHUNKS = 160 - C0_CHUNKS


@functools.partial(
    pl.kernel,
    mesh=_mesh,
    out_type=jax.ShapeDtypeStruct((NC, N_PAD, D), jnp.float32),
    scratch_types=[
        pltpu.VMEM((2, CHUNK), jnp.int32),
        pltpu.VMEM((CHUNK, D), jnp.float32),
        pltpu.VMEM_SHARED((ACC_ROWS, D), jnp.float32),
        pltpu.SemaphoreType.DMA,
    ],
)
def _sc_segsum_wide(x_hbm, edges_hbm, zeros_hbm, out_hbm, idx_v, rows_v, acc_sh,
                    sem):
    c = lax.axis_index("c")
    s = lax.axis_index("s")
    row0 = s * RPT
    nch = jnp.where(c == 0, C0_CHUNKS, C1_CHUNKS)
    cbase = jnp.where(c == 0, s * C0_CHUNKS, NS * C0_CHUNKS + s * C1_CHUNKS)

    @pl.when(s < NS - 1)
    def _():
        pltpu.sync_copy(zeros_hbm, acc_sh.at[pl.ds(row0, RPT)])

    @pl.when(s == NS - 1)
    def _():
        pltpu.sync_copy(zeros_hbm.at[pl.ds(0, LAST_RPT)],
                        acc_sh.at[pl.ds(row0, LAST_RPT)])

    plsc.subcore_barrier()

    def body(g, carry):
        off = (cbase + g) * CHUNK
        pltpu.sync_copy(edges_hbm.at[:, pl.ds(off, CHUNK)], idx_v)
        pltpu.async_copy(x_hbm.at[idx_v.at[0]], rows_v, sem).wait()
        pltpu.sync_copy(rows_v, acc_sh.at[idx_v.at[1]], add=True)
        return carry

    lax.fori_loop(0, nch, body, 0)
    plsc.subcore_barrier()

    @pl.when(s < NS - 1)
    def _():
        pltpu.sync_copy(acc_sh.at[pl.ds(row0, RPT)],
                        out_hbm.at[c, pl.ds(row0, RPT)])

    @pl.when(s == NS - 1)
    def _():
        pltpu.sync_copy(acc_sh.at[pl.ds(row0, LAST_RPT)],
                        out_hbm.at[c, pl.ds(row0, LAST_RPT)])


M_FLAT = 16384


@functools.partial(
    pl.kernel,
    mesh=_mesh,
    out_type=jax.ShapeDtypeStruct((NW * M_FLAT,), jnp.float32),
    scratch_types=[
        pltpu.VMEM((2, EPW), jnp.int32),
        pltpu.VMEM((M_FLAT,), jnp.float32),
        pltpu.VMEM((M_FLAT,), jnp.float32),
    ],
    compiler_params=pltpu.CompilerParams(needs_layout_passes=False),
)
def _sc_segsum_narrow(m_hbm, edges_hbm, zeros_hbm, out_hbm, eb_v, m_v, part_v):
    c = lax.axis_index("c")
    s = lax.axis_index("s")
    w = c * NS + s
    pltpu.sync_copy(edges_hbm.at[:, pl.ds(w * EPW, EPW)], eb_v)
    pltpu.sync_copy(m_hbm, m_v)
    pltpu.sync_copy(zeros_hbm, part_v)

    def body(i, carry):
        s16 = eb_v[0, pl.ds(i * 16, 16)]
        d16 = eb_v[1, pl.ds(i * 16, 16)]
        v = plsc.load_gather(m_v, [s16])
        plsc.addupdate_scatter(part_v, [d16], v)
        return carry

    lax.fori_loop(0, EPW // 16, body, 0)
    pltpu.sync_copy(part_v, out_hbm.at[pl.ds(w * M_FLAT, M_FLAT)])


def _tc_finish_body(parts_ref, out_ref):
    out_ref[...] = jnp.maximum(jnp.sum(parts_ref[...], axis=0), 0.0)


_tc_finish = pl.pallas_call(
    _tc_finish_body,
    grid=(M_FLAT // (8 * D),),
    in_specs=[pl.BlockSpec((NW, 8, D), lambda i: (0, i, 0))],
    out_specs=pl.BlockSpec((8, D), lambda i: (i, 0)),
    out_shape=jax.ShapeDtypeStruct((M_FLAT // D, D), jnp.float32),
)


def _tc_body(p0_ref, p1_ref, w1_ref, w2_ref, out_ref):
    sacc = p0_ref[...] + p1_ref[...]
    h = jnp.maximum(
        jax.lax.dot(sacc, w1_ref[...], preferred_element_type=jnp.float32), 0.0
    )
    out_ref[...] = jax.lax.dot(h, w2_ref[...], preferred_element_type=jnp.float32)


_TC_BLOCK = 128
_tc_matmul = pl.pallas_call(
    _tc_body,
    grid=(N_PAD // _TC_BLOCK,),
    in_specs=[
        pl.BlockSpec((_TC_BLOCK, D), lambda i: (i, 0)),
        pl.BlockSpec((_TC_BLOCK, D), lambda i: (i, 0)),
        pl.BlockSpec((D, D), lambda i: (0, 0)),
        pl.BlockSpec((D, OUT_W), lambda i: (0, 0)),
    ],
    out_specs=pl.BlockSpec((_TC_BLOCK, OUT_W), lambda i: (i, 0)),
    out_shape=jax.ShapeDtypeStruct((N_PAD, OUT_W), jnp.float32),
)


def kernel(x, edge_index, batch, W1, W2):
    pad = E_PAD - E_EDGES
    src = jnp.concatenate([edge_index[0], jnp.zeros((pad,), jnp.int32)])
    dst = jnp.concatenate([edge_index[1], jnp.full((pad,), N_NODES, jnp.int32)])
    edges = jnp.stack([src, dst])
    z_wide = jnp.zeros((RPT, D), jnp.float32)
    z_flat = jnp.zeros((M_FLAT,), jnp.float32)
    w2p = jnp.pad(W2, ((0, 0), (0, OUT_W - 1)))

    p = _sc_segsum_wide(x, edges, z_wide)
    m = _tc_matmul(p[0], p[1], W1, w2p)
    m_flat = jnp.pad(m[:, 0], (0, M_FLAT - N_PAD))
    parts = _sc_segsum_narrow(m_flat, edges, z_flat)
    out = _tc_finish(parts.reshape(NW, M_FLAT // D, D))
    return out.reshape(-1)[:N_NODES].reshape(N_NODES, 1)


# speedup vs baseline: 1.1633x; 1.1633x over previous
import functools

import jax
import jax.numpy as jnp
from jax import lax
from jax.experimental import pallas as pl
from jax.experimental.pallas import tpu as pltpu
from jax.experimental.pallas import tpu_sc as plsc

N_NODES = 10000
E_EDGES = 320000
D = 128

NC = 2
NS = 16
NW = NC * NS

CHUNK = 128
N_CHUNKS = 80
EPW = CHUNK * N_CHUNKS
E_PAD = EPW * NW
N_PAD = 10112
ACC_ROWS = 10008
RPT = 632
LAST_RPT = ACC_ROWS - 15 * RPT
OUT_W = 16

_mesh = plsc.VectorSubcoreMesh(core_axis_name="c", subcore_axis_name="s")


C0_CHUNKS = 104
C1_CHUNKS = 160 - C0_CHUNKS


@functools.partial(
    pl.kernel,
    mesh=_mesh,
    out_type=jax.ShapeDtypeStruct((NC, N_PAD, D), jnp.float32),
    scratch_types=[
        pltpu.VMEM((2, CHUNK), jnp.int32),
        pltpu.VMEM((CHUNK, D), jnp.float32),
        pltpu.VMEM_SHARED((ACC_ROWS, D), jnp.float32),
        pltpu.SemaphoreType.DMA,
    ],
)
def _sc_segsum_wide(x_hbm, edges_hbm, zeros_hbm, out_hbm, idx_v, rows_v, acc_sh,
                    sem):
    c = lax.axis_index("c")
    s = lax.axis_index("s")
    row0 = s * RPT
    nch = jnp.where(c == 0, C0_CHUNKS, C1_CHUNKS)
    cbase = jnp.where(c == 0, s * C0_CHUNKS, NS * C0_CHUNKS + s * C1_CHUNKS)

    @pl.when(s < NS - 1)
    def _():
        pltpu.sync_copy(zeros_hbm, acc_sh.at[pl.ds(row0, RPT)])

    @pl.when(s == NS - 1)
    def _():
        pltpu.sync_copy(zeros_hbm.at[pl.ds(0, LAST_RPT)],
                        acc_sh.at[pl.ds(row0, LAST_RPT)])

    plsc.subcore_barrier()

    def body(g, carry):
        off = (cbase + g) * CHUNK
        pltpu.sync_copy(edges_hbm.at[:, pl.ds(off, CHUNK)], idx_v)
        pltpu.async_copy(x_hbm.at[idx_v.at[0]], rows_v, sem).wait()
        pltpu.sync_copy(rows_v, acc_sh.at[idx_v.at[1]], add=True)
        return carry

    lax.fori_loop(0, nch, body, 0)
    plsc.subcore_barrier()

    @pl.when(s < NS - 1)
    def _():
        pltpu.sync_copy(acc_sh.at[pl.ds(row0, RPT)],
                        out_hbm.at[c, pl.ds(row0, RPT)])

    @pl.when(s == NS - 1)
    def _():
        pltpu.sync_copy(acc_sh.at[pl.ds(row0, LAST_RPT)],
                        out_hbm.at[c, pl.ds(row0, LAST_RPT)])


M_FLAT = 16384


@functools.partial(
    pl.kernel,
    mesh=_mesh,
    out_type=jax.ShapeDtypeStruct((NW * M_FLAT,), jnp.float32),
    scratch_types=[
        pltpu.VMEM((2, EPW), jnp.int32),
        pltpu.VMEM((M_FLAT,), jnp.float32),
        pltpu.VMEM((M_FLAT,), jnp.float32),
    ],
    compiler_params=pltpu.CompilerParams(needs_layout_passes=False),
)
def _sc_segsum_narrow(m_hbm, edges_hbm, zeros_hbm, out_hbm, eb_v, m_v, part_v):
    c = lax.axis_index("c")
    s = lax.axis_index("s")
    w = c * NS + s
    pltpu.sync_copy(edges_hbm.at[:, pl.ds(w * EPW, EPW)], eb_v)
    pltpu.sync_copy(m_hbm, m_v)
    pltpu.sync_copy(zeros_hbm, part_v)

    def body(i, carry):
        s16 = eb_v[0, pl.ds(i * 16, 16)]
        d16 = eb_v[1, pl.ds(i * 16, 16)]
        v = plsc.load_gather(m_v, [s16])
        plsc.addupdate_scatter(part_v, [d16], v)
        return carry

    lax.fori_loop(0, EPW // 16, body, 0)
    pltpu.sync_copy(part_v, out_hbm.at[pl.ds(w * M_FLAT, M_FLAT)])


def _tc_finish_body(parts_ref, out_ref):
    out_ref[...] = jnp.maximum(jnp.sum(parts_ref[...], axis=0), 0.0)


_tc_finish = pl.pallas_call(
    _tc_finish_body,
    grid=(M_FLAT // (8 * D),),
    in_specs=[pl.BlockSpec((NW, 8, D), lambda i: (0, i, 0))],
    out_specs=pl.BlockSpec((8, D), lambda i: (i, 0)),
    out_shape=jax.ShapeDtypeStruct((M_FLAT // D, D), jnp.float32),
)


def _tc_body(p0_ref, p1_ref, w1_ref, w2_ref, out_ref):
    sacc = p0_ref[...] + p1_ref[...]
    h = jnp.maximum(
        jax.lax.dot(sacc, w1_ref[...], preferred_element_type=jnp.float32), 0.0
    )
    out_ref[...] = jax.lax.dot(h, w2_ref[...], preferred_element_type=jnp.float32)


_TC_BLOCK = 128
_tc_matmul = pl.pallas_call(
    _tc_body,
    grid=(N_PAD // _TC_BLOCK,),
    in_specs=[
        pl.BlockSpec((_TC_BLOCK, D), lambda i: (i, 0)),
        pl.BlockSpec((_TC_BLOCK, D), lambda i: (i, 0)),
        pl.BlockSpec((D, D), lambda i: (0, 0)),
        pl.BlockSpec((D, OUT_W), lambda i: (0, 0)),
    ],
    out_specs=pl.BlockSpec((_TC_BLOCK, OUT_W), lambda i: (i, 0)),
    out_shape=jax.ShapeDtypeStruct((N_PAD, OUT_W), jnp.float32),
)


def kernel(x, edge_index, batch, W1, W2):
    pad = E_PAD - E_EDGES
    src = jnp.concatenate([edge_index[0], jnp.zeros((pad,), jnp.int32)])
    dst = jnp.concatenate([edge_index[1], jnp.full((pad,), N_NODES, jnp.int32)])
    edges = jnp.stack([src, dst])
    z_wide = jnp.zeros((RPT, D), jnp.float32)
    z_flat = jnp.zeros((M_FLAT,), jnp.float32)
    w2p = jnp.pad(W2, ((0, 0), (0, OUT_W - 1)))

    p = _sc_segsum_wide(x, edges, z_wide)
    m = _tc_matmul(p[0], p[1], W1, w2p)
    m_flat = jnp.pad(m[:, 0], (0, M_FLAT - N_PAD))
    parts = _sc_segsum_narrow(m_flat, edges, z_flat)
    out = _tc_finish(parts.reshape(NW, M_FLAT // D, D))
    return out.reshape(-1)[:N_NODES].reshape(N_NODES, 1)


# speedup vs baseline: 2.3899x; 2.0544x over previous
import functools

import jax
import jax.numpy as jnp
from jax import lax
from jax.experimental import pallas as pl
from jax.experimental.pallas import tpu as pltpu
from jax.experimental.pallas import tpu_sc as plsc

N_NODES = 10000
E_EDGES = 320000
D = 128

NC = 2
NS = 16
NW = NC * NS

CHUNK = 128
N_CHUNKS = 80
EPW = CHUNK * N_CHUNKS
E_PAD = EPW * NW
N_PAD = 10112
ACC_ROWS = 10008
RPT = 632
LAST_RPT = ACC_ROWS - 15 * RPT
OUT_W = 16

_mesh = plsc.VectorSubcoreMesh(core_axis_name="c", subcore_axis_name="s")


C0_CHUNKS = 80
C1_CHUNKS = 160 - C0_CHUNKS


@functools.partial(
    pl.kernel,
    mesh=_mesh,
    out_type=jax.ShapeDtypeStruct((NC, N_PAD, D), jnp.float32),
    scratch_types=[
        pltpu.VMEM((2, CHUNK), jnp.int32),
        pltpu.VMEM((CHUNK, D), jnp.float32),
        pltpu.VMEM_SHARED((ACC_ROWS, D), jnp.float32),
        pltpu.SemaphoreType.DMA,
    ],
)
def _sc_segsum_wide(x_hbm, edges_hbm, zeros_hbm, out_hbm, idx_v, rows_v, acc_sh,
                    sem):
    c = lax.axis_index("c")
    s = lax.axis_index("s")
    row0 = s * RPT
    nch = jnp.where(c == 0, C0_CHUNKS, C1_CHUNKS)
    cbase = jnp.where(c == 0, s * C0_CHUNKS, NS * C0_CHUNKS + s * C1_CHUNKS)

    @pl.when(s < NS - 1)
    def _():
        pltpu.sync_copy(zeros_hbm, acc_sh.at[pl.ds(row0, RPT)])

    @pl.when(s == NS - 1)
    def _():
        pltpu.sync_copy(zeros_hbm.at[pl.ds(0, LAST_RPT)],
                        acc_sh.at[pl.ds(row0, LAST_RPT)])

    plsc.subcore_barrier()

    def body(g, carry):
        off = (cbase + g) * CHUNK
        pltpu.sync_copy(edges_hbm.at[:, pl.ds(off, CHUNK)], idx_v)
        pltpu.async_copy(x_hbm.at[idx_v.at[0]], rows_v, sem).wait()
        pltpu.sync_copy(rows_v, acc_sh.at[idx_v.at[1]], add=True)
        return carry

    lax.fori_loop(0, nch, body, 0)
    plsc.subcore_barrier()

    @pl.when(s < NS - 1)
    def _():
        pltpu.sync_copy(acc_sh.at[pl.ds(row0, RPT)],
                        out_hbm.at[c, pl.ds(row0, RPT)])

    @pl.when(s == NS - 1)
    def _():
        pltpu.sync_copy(acc_sh.at[pl.ds(row0, LAST_RPT)],
                        out_hbm.at[c, pl.ds(row0, LAST_RPT)])


M_FLAT = 16384


@functools.partial(
    pl.kernel,
    mesh=_mesh,
    out_type=jax.ShapeDtypeStruct((NW * M_FLAT,), jnp.float32),
    scratch_types=[
        pltpu.VMEM((2, EPW), jnp.int32),
        pltpu.VMEM((M_FLAT,), jnp.float32),
        pltpu.VMEM((M_FLAT,), jnp.float32),
    ],
    compiler_params=pltpu.CompilerParams(needs_layout_passes=False),
)
def _sc_segsum_narrow(m_hbm, edges_hbm, zeros_hbm, out_hbm, eb_v, m_v, part_v):
    c = lax.axis_index("c")
    s = lax.axis_index("s")
    w = c * NS + s
    pltpu.sync_copy(edges_hbm.at[:, pl.ds(w * EPW, EPW)], eb_v)
    pltpu.sync_copy(m_hbm, m_v)
    pltpu.sync_copy(zeros_hbm, part_v)

    def body(i, carry):
        s16 = eb_v[0, pl.ds(i * 16, 16)]
        d16 = eb_v[1, pl.ds(i * 16, 16)]
        v = plsc.load_gather(m_v, [s16])
        plsc.addupdate_scatter(part_v, [d16], v)
        return carry

    lax.fori_loop(0, EPW // 16, body, 0)
    pltpu.sync_copy(part_v, out_hbm.at[pl.ds(w * M_FLAT, M_FLAT)])


def _tc_finish_body(parts_ref, out_ref):
    out_ref[...] = jnp.maximum(jnp.sum(parts_ref[...], axis=0), 0.0)


_tc_finish = pl.pallas_call(
    _tc_finish_body,
    grid=(M_FLAT // (8 * D),),
    in_specs=[pl.BlockSpec((NW, 8, D), lambda i: (0, i, 0))],
    out_specs=pl.BlockSpec((8, D), lambda i: (i, 0)),
    out_shape=jax.ShapeDtypeStruct((M_FLAT // D, D), jnp.float32),
)


def _tc_body(p0_ref, p1_ref, w1_ref, w2_ref, out_ref):
    sacc = p0_ref[...] + p1_ref[...]
    h = jnp.maximum(
        jax.lax.dot(sacc, w1_ref[...], preferred_element_type=jnp.float32), 0.0
    )
    out_ref[...] = jax.lax.dot(h, w2_ref[...], preferred_element_type=jnp.float32)


_TC_BLOCK = 128
_tc_matmul = pl.pallas_call(
    _tc_body,
    grid=(N_PAD // _TC_BLOCK,),
    in_specs=[
        pl.BlockSpec((_TC_BLOCK, D), lambda i: (i, 0)),
        pl.BlockSpec((_TC_BLOCK, D), lambda i: (i, 0)),
        pl.BlockSpec((D, D), lambda i: (0, 0)),
        pl.BlockSpec((D, OUT_W), lambda i: (0, 0)),
    ],
    out_specs=pl.BlockSpec((_TC_BLOCK, OUT_W), lambda i: (i, 0)),
    out_shape=jax.ShapeDtypeStruct((N_PAD, OUT_W), jnp.float32),
)


def kernel(x, edge_index, batch, W1, W2):
    pad = E_PAD - E_EDGES
    pad_ids = jnp.arange(pad, dtype=jnp.int32)
    src = jnp.concatenate([edge_index[0], N_PAD + (pad_ids % 16)])
    dst = jnp.concatenate([edge_index[1], pad_ids % N_NODES])
    edges = jnp.stack([src, dst])
    x_pad = jnp.concatenate([x, jnp.zeros((N_PAD + 16 - N_NODES, D), jnp.float32)])
    z_wide = jnp.zeros((RPT, D), jnp.float32)
    z_flat = jnp.zeros((M_FLAT,), jnp.float32)
    w2p = jnp.pad(W2, ((0, 0), (0, OUT_W - 1)))

    p = _sc_segsum_wide(x_pad, edges, z_wide)
    m = _tc_matmul(p[0], p[1], W1, w2p)
    m_flat = jnp.pad(m[:, 0], (0, M_FLAT - N_PAD))
    parts = _sc_segsum_narrow(m_flat, edges, z_flat)
    out = _tc_finish(parts.reshape(NW, M_FLAT // D, D))
    return out.reshape(-1)[:N_NODES].reshape(N_NODES, 1)


# speedup vs baseline: 2.9153x; 1.2198x over previous
import functools

import jax
import jax.numpy as jnp
from jax import lax
from jax.experimental import pallas as pl
from jax.experimental.pallas import tpu as pltpu
from jax.experimental.pallas import tpu_sc as plsc

N_NODES = 10000
E_EDGES = 320000
D = 128

NC = 2
NS = 16
NW = NC * NS

CHUNK = 128
TOT_CHUNKS = E_EDGES // CHUNK
BASE_CH = TOT_CHUNKS // NW
EXTRA_CH = TOT_CHUNKS - BASE_CH * NW
EPW2 = E_EDGES // NW
N_PAD = 10112
ACC_ROWS = 10008
RPT = 632
LAST_RPT = ACC_ROWS - 15 * RPT
OUT_W = 16

_mesh = plsc.VectorSubcoreMesh(core_axis_name="c", subcore_axis_name="s")


@functools.partial(
    pl.kernel,
    mesh=_mesh,
    out_type=jax.ShapeDtypeStruct((NC, N_PAD, D), jnp.float32),
    scratch_types=[
        pltpu.VMEM((2, CHUNK), jnp.int32),
        pltpu.VMEM((CHUNK, D), jnp.float32),
        pltpu.VMEM_SHARED((ACC_ROWS, D), jnp.float32),
        pltpu.SemaphoreType.DMA,
    ],
)
def _sc_segsum_wide(x_hbm, edges_hbm, zeros_hbm, out_hbm, idx_v, rows_v, acc_sh,
                    sem):
    c = lax.axis_index("c")
    s = lax.axis_index("s")
    w = c * NS + s
    row0 = s * RPT
    nch = jnp.where(w < EXTRA_CH, BASE_CH + 1, BASE_CH)
    cbase = w * BASE_CH + jnp.minimum(w, EXTRA_CH)

    @pl.when(s < NS - 1)
    def _():
        pltpu.sync_copy(zeros_hbm, acc_sh.at[pl.ds(row0, RPT)])

    @pl.when(s == NS - 1)
    def _():
        pltpu.sync_copy(zeros_hbm.at[pl.ds(0, LAST_RPT)],
                        acc_sh.at[pl.ds(row0, LAST_RPT)])

    plsc.subcore_barrier()

    def body(g, carry):
        off = (cbase + g) * CHUNK
        pltpu.sync_copy(edges_hbm.at[:, pl.ds(off, CHUNK)], idx_v)
        pltpu.async_copy(x_hbm.at[idx_v.at[0]], rows_v, sem).wait()
        pltpu.sync_copy(rows_v, acc_sh.at[idx_v.at[1]], add=True)
        return carry

    lax.fori_loop(0, nch, body, 0)
    plsc.subcore_barrier()

    @pl.when(s < NS - 1)
    def _():
        pltpu.sync_copy(acc_sh.at[pl.ds(row0, RPT)],
                        out_hbm.at[c, pl.ds(row0, RPT)])

    @pl.when(s == NS - 1)
    def _():
        pltpu.sync_copy(acc_sh.at[pl.ds(row0, LAST_RPT)],
                        out_hbm.at[c, pl.ds(row0, LAST_RPT)])


M_FLAT = 16384


@functools.partial(
    pl.kernel,
    mesh=_mesh,
    out_type=jax.ShapeDtypeStruct((NW * M_FLAT,), jnp.float32),
    scratch_types=[
        pltpu.VMEM((2, BASE_CH * CHUNK), jnp.int32),
        pltpu.VMEM((2, CHUNK), jnp.int32),
        pltpu.VMEM((M_FLAT,), jnp.float32),
        pltpu.VMEM((M_FLAT,), jnp.float32),
    ],
    compiler_params=pltpu.CompilerParams(needs_layout_passes=False),
)
def _sc_segsum_narrow(m_hbm, edges_hbm, zeros_hbm, out_hbm, eb_v, ex_v, m_v,
                      part_v):
    c = lax.axis_index("c")
    s = lax.axis_index("s")
    w = c * NS + s
    cbase = w * BASE_CH + jnp.minimum(w, EXTRA_CH)
    pltpu.sync_copy(edges_hbm.at[:, pl.ds(cbase * CHUNK, BASE_CH * CHUNK)], eb_v)

    @pl.when(w < EXTRA_CH)
    def _():
        pltpu.sync_copy(
            edges_hbm.at[:, pl.ds((cbase + BASE_CH) * CHUNK, CHUNK)], ex_v)

    pltpu.sync_copy(m_hbm, m_v)
    pltpu.sync_copy(zeros_hbm, part_v)

    def body(i, carry):
        s16 = eb_v[0, pl.ds(i * 16, 16)]
        d16 = eb_v[1, pl.ds(i * 16, 16)]
        v = plsc.load_gather(m_v, [s16])
        plsc.addupdate_scatter(part_v, [d16], v)
        return carry

    lax.fori_loop(0, (BASE_CH * CHUNK) // 16, body, 0)

    @pl.when(w < EXTRA_CH)
    def _():
        def xbody(i, carry):
            s16 = ex_v[0, pl.ds(i * 16, 16)]
            d16 = ex_v[1, pl.ds(i * 16, 16)]
            v = plsc.load_gather(m_v, [s16])
            plsc.addupdate_scatter(part_v, [d16], v)
            return carry

        lax.fori_loop(0, CHUNK // 16, xbody, 0)

    pltpu.sync_copy(part_v, out_hbm.at[pl.ds(w * M_FLAT, M_FLAT)])


def _tc_finish_body(parts_ref, out_ref):
    out_ref[...] = jnp.maximum(jnp.sum(parts_ref[...], axis=0), 0.0)


_tc_finish = pl.pallas_call(
    _tc_finish_body,
    grid=(M_FLAT // (8 * D),),
    in_specs=[pl.BlockSpec((NW, 8, D), lambda i: (0, i, 0))],
    out_specs=pl.BlockSpec((8, D), lambda i: (i, 0)),
    out_shape=jax.ShapeDtypeStruct((M_FLAT // D, D), jnp.float32),
)


def _tc_body(p0_ref, p1_ref, w1_ref, w2_ref, out_ref):
    sacc = p0_ref[...] + p1_ref[...]
    h = jnp.maximum(
        jax.lax.dot(sacc, w1_ref[...], preferred_element_type=jnp.float32), 0.0
    )
    out_ref[...] = jax.lax.dot(h, w2_ref[...], preferred_element_type=jnp.float32)


_TC_BLOCK = 1264
_tc_matmul = pl.pallas_call(
    _tc_body,
    grid=(N_PAD // _TC_BLOCK,),
    in_specs=[
        pl.BlockSpec((_TC_BLOCK, D), lambda i: (i, 0)),
        pl.BlockSpec((_TC_BLOCK, D), lambda i: (i, 0)),
        pl.BlockSpec((D, D), lambda i: (0, 0)),
        pl.BlockSpec((D, OUT_W), lambda i: (0, 0)),
    ],
    out_specs=pl.BlockSpec((_TC_BLOCK, OUT_W), lambda i: (i, 0)),
    out_shape=jax.ShapeDtypeStruct((N_PAD, OUT_W), jnp.float32),
)


def kernel(x, edge_index, batch, W1, W2):
    edges = edge_index
    z_wide = jnp.zeros((RPT, D), jnp.float32)
    z_flat = jnp.zeros((M_FLAT,), jnp.float32)
    w2p = jnp.pad(W2, ((0, 0), (0, OUT_W - 1)))

    p = _sc_segsum_wide(x, edges, z_wide)
    m = _tc_matmul(p[0], p[1], W1, w2p)
    m_flat = jnp.pad(m[:, 0], (0, M_FLAT - N_PAD))
    parts = _sc_segsum_narrow(m_flat, edges, z_flat)
    out = _tc_finish(parts.reshape(NW, M_FLAT // D, D))
    return out.reshape(-1)[:N_NODES].reshape(N_NODES, 1)


# speedup vs baseline: 3.3657x; 1.1545x over previous
import functools

import jax
import jax.numpy as jnp
from jax import lax
from jax.experimental import pallas as pl
from jax.experimental.pallas import tpu as pltpu
from jax.experimental.pallas import tpu_sc as plsc

N_NODES = 10000
E_EDGES = 320000
D = 128

NC = 2
NS = 16
NW = NC * NS

CHUNK = 128
TOT_CHUNKS = E_EDGES // CHUNK
BASE_CH = TOT_CHUNKS // NW
EXTRA_CH = TOT_CHUNKS - BASE_CH * NW
EPW2 = E_EDGES // NW
N_PAD = 10112
ACC_ROWS = 10008
RPT = 632
LAST_RPT = ACC_ROWS - 15 * RPT
OUT_W = 16

_mesh = plsc.VectorSubcoreMesh(core_axis_name="c", subcore_axis_name="s")


@functools.partial(
    pl.kernel,
    mesh=_mesh,
    out_type=jax.ShapeDtypeStruct((NC, N_PAD, D), jnp.float32),
    scratch_types=[
        pltpu.VMEM((2, 2, CHUNK), jnp.int32),
        pltpu.VMEM((CHUNK, D), jnp.float32),
        pltpu.VMEM_SHARED((ACC_ROWS, D), jnp.float32),
        pltpu.SemaphoreType.DMA,
        pltpu.SemaphoreType.DMA,
    ],
)
def _sc_segsum_wide(x_hbm, edges_hbm, zeros_hbm, out_hbm, idx_v, rows_v, acc_sh,
                    sem, isem):
    c = lax.axis_index("c")
    s = lax.axis_index("s")
    w = c * NS + s
    row0 = s * RPT
    nch = jnp.where(w < EXTRA_CH, BASE_CH + 1, BASE_CH)
    cbase = w * BASE_CH + jnp.minimum(w, EXTRA_CH)

    @pl.when(s < NS - 1)
    def _():
        pltpu.sync_copy(zeros_hbm, acc_sh.at[pl.ds(row0, RPT)])

    @pl.when(s == NS - 1)
    def _():
        pltpu.sync_copy(zeros_hbm.at[pl.ds(0, LAST_RPT)],
                        acc_sh.at[pl.ds(row0, LAST_RPT)])

    pltpu.sync_copy(edges_hbm.at[:, pl.ds(cbase * CHUNK, CHUNK)], idx_v.at[0])
    plsc.subcore_barrier()

    def body(g, carry):
        q = lax.rem(g, 2)

        @pl.when(g + 1 < nch)
        def _():
            pltpu.async_copy(
                edges_hbm.at[:, pl.ds((cbase + g + 1) * CHUNK, CHUNK)],
                idx_v.at[1 - q], isem)

        pltpu.async_copy(x_hbm.at[idx_v.at[q, 0]], rows_v, sem).wait()
        pltpu.sync_copy(rows_v, acc_sh.at[idx_v.at[q, 1]], add=True)

        @pl.when(g + 1 < nch)
        def _():
            pltpu.make_async_copy(
                edges_hbm.at[:, pl.ds((cbase + g + 1) * CHUNK, CHUNK)],
                idx_v.at[1 - q], isem).wait()

        return carry

    lax.fori_loop(0, nch, body, 0)
    plsc.subcore_barrier()

    @pl.when(s < NS - 1)
    def _():
        pltpu.sync_copy(acc_sh.at[pl.ds(row0, RPT)],
                        out_hbm.at[c, pl.ds(row0, RPT)])

    @pl.when(s == NS - 1)
    def _():
        pltpu.sync_copy(acc_sh.at[pl.ds(row0, LAST_RPT)],
                        out_hbm.at[c, pl.ds(row0, LAST_RPT)])


M_FLAT = 16384


@functools.partial(
    pl.kernel,
    mesh=_mesh,
    out_type=jax.ShapeDtypeStruct((NW * M_FLAT,), jnp.float32),
    scratch_types=[
        pltpu.VMEM((2, BASE_CH * CHUNK), jnp.int32),
        pltpu.VMEM((2, CHUNK), jnp.int32),
        pltpu.VMEM((M_FLAT,), jnp.float32),
        pltpu.VMEM((M_FLAT,), jnp.float32),
    ],
    compiler_params=pltpu.CompilerParams(needs_layout_passes=False),
)
def _sc_segsum_narrow(m_hbm, edges_hbm, zeros_hbm, out_hbm, eb_v, ex_v, m_v,
                      part_v):
    c = lax.axis_index("c")
    s = lax.axis_index("s")
    w = c * NS + s
    cbase = w * BASE_CH + jnp.minimum(w, EXTRA_CH)
    pltpu.sync_copy(edges_hbm.at[:, pl.ds(cbase * CHUNK, BASE_CH * CHUNK)], eb_v)

    @pl.when(w < EXTRA_CH)
    def _():
        pltpu.sync_copy(
            edges_hbm.at[:, pl.ds((cbase + BASE_CH) * CHUNK, CHUNK)], ex_v)

    pltpu.sync_copy(m_hbm, m_v)
    pltpu.sync_copy(zeros_hbm, part_v)

    def body(i, carry):
        s16 = eb_v[0, pl.ds(i * 16, 16)]
        d16 = eb_v[1, pl.ds(i * 16, 16)]
        v = plsc.load_gather(m_v, [s16])
        plsc.addupdate_scatter(part_v, [d16], v)
        return carry

    lax.fori_loop(0, (BASE_CH * CHUNK) // 16, body, 0)

    @pl.when(w < EXTRA_CH)
    def _():
        def xbody(i, carry):
            s16 = ex_v[0, pl.ds(i * 16, 16)]
            d16 = ex_v[1, pl.ds(i * 16, 16)]
            v = plsc.load_gather(m_v, [s16])
            plsc.addupdate_scatter(part_v, [d16], v)
            return carry

        lax.fori_loop(0, CHUNK // 16, xbody, 0)

    pltpu.sync_copy(part_v, out_hbm.at[pl.ds(w * M_FLAT, M_FLAT)])


def _tc_finish_body(parts_ref, out_ref):
    out_ref[...] = jnp.maximum(jnp.sum(parts_ref[...], axis=0), 0.0)


_tc_finish = pl.pallas_call(
    _tc_finish_body,
    grid=(M_FLAT // (8 * D),),
    in_specs=[pl.BlockSpec((NW, 8, D), lambda i: (0, i, 0))],
    out_specs=pl.BlockSpec((8, D), lambda i: (i, 0)),
    out_shape=jax.ShapeDtypeStruct((M_FLAT // D, D), jnp.float32),
)


def _tc_body(p0_ref, p1_ref, w1_ref, w2_ref, out_ref):
    sacc = p0_ref[...] + p1_ref[...]
    h = jnp.maximum(
        jax.lax.dot(sacc, w1_ref[...], preferred_element_type=jnp.float32), 0.0
    )
    out_ref[...] = jax.lax.dot(h, w2_ref[...], preferred_element_type=jnp.float32)


_TC_BLOCK = 1264
_tc_matmul = pl.pallas_call(
    _tc_body,
    grid=(N_PAD // _TC_BLOCK,),
    in_specs=[
        pl.BlockSpec((_TC_BLOCK, D), lambda i: (i, 0)),
        pl.BlockSpec((_TC_BLOCK, D), lambda i: (i, 0)),
        pl.BlockSpec((D, D), lambda i: (0, 0)),
        pl.BlockSpec((D, OUT_W), lambda i: (0, 0)),
    ],
    out_specs=pl.BlockSpec((_TC_BLOCK, OUT_W), lambda i: (i, 0)),
    out_shape=jax.ShapeDtypeStruct((N_PAD, OUT_W), jnp.float32),
)


def kernel(x, edge_index, batch, W1, W2):
    edges = edge_index
    z_wide = jnp.zeros((RPT, D), jnp.float32)
    z_flat = jnp.zeros((M_FLAT,), jnp.float32)
    w2p = jnp.pad(W2, ((0, 0), (0, OUT_W - 1)))

    p = _sc_segsum_wide(x, edges, z_wide)
    m = _tc_matmul(p[0], p[1], W1, w2p)
    m_flat = jnp.pad(m[:, 0], (0, M_FLAT - N_PAD))
    parts = _sc_segsum_narrow(m_flat, edges, z_flat)
    out = _tc_finish(parts.reshape(NW, M_FLAT // D, D))
    return out.reshape(-1)[:N_NODES].reshape(N_NODES, 1)
